# baseline (speedup 1.0000x reference)
import functools

import jax
import jax.numpy as jnp
from jax import lax
from jax.experimental import pallas as pl
from jax.experimental.pallas import tpu as pltpu
from jax.experimental.pallas import tpu_sc as plsc

B, T, C = 4096, 50, 20
ED = 16
NC, NS = 2, 16
NW = NC * NS
BP = B // NW


def _tree_sum(vals):
    while len(vals) > 1:
        nxt = [vals[i] + vals[i + 1] for i in range(0, len(vals) - 1, 2)]
        if len(vals) % 2:
            nxt.append(vals[-1])
        vals = nxt
    return vals[0]


def _emb_body(idx_hbm, table_hbm, out_hbm,
              idx0, idx1, rows0, rows1, out0, out1,
              gsem0, gsem1, osem0, osem1):
    wid = lax.axis_index("s") * NC + lax.axis_index("c")
    b0 = wid * BP
    idx_b = (idx0, idx1)
    rows_b = (rows0, rows1)
    out_b = (out0, out1)
    gsem = (gsem0, gsem1)
    osem = (osem0, osem1)

    def _fire_gather(b, t):
        pltpu.sync_copy(idx_hbm.at[:, pl.ds(t, 1), pl.ds(b0, BP)], idx_b[b])
        for c in range(C):
            pltpu.async_copy(
                table_hbm.at[idx_b[b].at[c, 0]], rows_b[b].at[c], gsem[b])

    def _wait_gather(b):
        for c in range(C):
            pltpu.make_async_copy(
                table_hbm.at[idx_b[b].at[c, 0]], rows_b[b].at[c], gsem[b]).wait()

    for b in range(2):
        _fire_gather(b, b)

    @pl.loop(0, T, step=2)
    def _chunk(t0):
        for b in range(2):
            t = t0 + b
            _wait_gather(b)
            @pl.when(t >= 2)
            def _():
                pltpu.make_async_copy(
                    out_b[b], out_hbm.at[pl.ds(b0, BP), pl.ds(0, 1)],
                    osem[b]).wait()

            @pl.loop(0, BP)
            def _pos(p):
                out_b[b][p, 0] = _tree_sum([rows_b[b][c, p] for c in range(C)])

            pltpu.async_copy(
                out_b[b], out_hbm.at[pl.ds(b0, BP), pl.ds(t, 1)], osem[b])

            @pl.when(t + 2 < T)
            def _():
                _fire_gather(b, t + 2)

    for b in range(2):
        pltpu.make_async_copy(
            out_b[b], out_hbm.at[pl.ds(b0, BP), pl.ds(0, 1)], osem[b]).wait()


def _repack_body(x_ref, o_ref):
    o_ref[...] = x_ref[...]


def _repack_idx(idx_ctb):
    return pl.pallas_call(
        _repack_body,
        grid=(C,),
        in_specs=[pl.BlockSpec((1, T, B), lambda c: (c, 0, 0))],
        out_specs=pl.BlockSpec((1, T, B), lambda c: (c, 0, 0)),
        out_shape=jax.ShapeDtypeStruct((C, T, B), jnp.int32),
    )(idx_ctb)


@jax.jit
def _embed_sum(idx_ctb, embed_table):
    mesh = plsc.VectorSubcoreMesh(core_axis_name="c", subcore_axis_name="s")
    return pl.kernel(
        _emb_body,
        out_type=jax.ShapeDtypeStruct((B, T, ED), jnp.float32),
        mesh=mesh,
        compiler_params=pltpu.CompilerParams(use_tc_tiling_on_sc=False),
        scratch_types=[
            pltpu.VMEM((C, 1, BP), jnp.int32),
            pltpu.VMEM((C, 1, BP), jnp.int32),
            pltpu.VMEM((C, BP, ED), jnp.float32),
            pltpu.VMEM((C, BP, ED), jnp.float32),
            pltpu.VMEM((BP, 1, ED), jnp.float32),
            pltpu.VMEM((BP, 1, ED), jnp.float32),
            pltpu.SemaphoreType.DMA,
            pltpu.SemaphoreType.DMA,
            pltpu.SemaphoreType.DMA,
            pltpu.SemaphoreType.DMA,
        ],
    )(idx_ctb, embed_table)


def kernel(ContTensor, CatTensor, LabelTensor, MaskTensor, DoseTensor, TimeDiffTensor, VTensor, VancoClTensor, PtList, LengList, embed_table):
    idx_ctb = _repack_idx(CatTensor.transpose(2, 1, 0))
    emb = _embed_sum(idx_ctb, embed_table)
    outEmb = jnp.concatenate((emb, ContTensor), axis=2)
    return (outEmb, LabelTensor, LengList, MaskTensor, DoseTensor, TimeDiffTensor, VTensor, VancoClTensor, PtList)


# speedup vs baseline: 1.0072x; 1.0072x over previous
import functools

import jax
import jax.numpy as jnp
from jax import lax
from jax.experimental import pallas as pl
from jax.experimental.pallas import tpu as pltpu
from jax.experimental.pallas import tpu_sc as plsc

B, T, C = 4096, 50, 20
ED = 16
NC, NS = 2, 16
NW = NC * NS
BP = B // NW


def _tree_sum(vals):
    while len(vals) > 1:
        nxt = [vals[i] + vals[i + 1] for i in range(0, len(vals) - 1, 2)]
        if len(vals) % 2:
            nxt.append(vals[-1])
        vals = nxt
    return vals[0]


def _emb_body(idx_hbm, table_hbm, out_hbm,
              idx0, idx1, rows0, rows1, out0, out1,
              gsem0, gsem1, osem0, osem1):
    wid = lax.axis_index("s") * NC + lax.axis_index("c")
    b0 = wid * BP
    idx_b = (idx0, idx1)
    rows_b = (rows0, rows1)
    out_b = (out0, out1)
    gsem = (gsem0, gsem1)
    osem = (osem0, osem1)

    def _fire_gather(b, t):
        pltpu.sync_copy(idx_hbm.at[:, t // 8, wid, t % 8], idx_b[b])
        for c in range(C):
            pltpu.async_copy(
                table_hbm.at[idx_b[b].at[c]], rows_b[b].at[c], gsem[b])

    def _wait_gather(b):
        for c in range(C):
            pltpu.make_async_copy(
                table_hbm.at[idx_b[b].at[c]], rows_b[b].at[c], gsem[b]).wait()

    for b in range(2):
        _fire_gather(b, b)

    @pl.loop(0, T, step=2)
    def _chunk(t0):
        for b in range(2):
            t = t0 + b
            _wait_gather(b)
            @pl.when(t >= 2)
            def _():
                pltpu.make_async_copy(
                    out_b[b], out_hbm.at[pl.ds(b0, BP), pl.ds(0, 1)],
                    osem[b]).wait()

            @pl.loop(0, BP)
            def _pos(p):
                out_b[b][p, 0] = _tree_sum([rows_b[b][c, p] for c in range(C)])

            pltpu.async_copy(
                out_b[b], out_hbm.at[pl.ds(b0, BP), pl.ds(t, 1)], osem[b])

            @pl.when(t + 2 < T)
            def _():
                _fire_gather(b, t + 2)

    for b in range(2):
        pltpu.make_async_copy(
            out_b[b], out_hbm.at[pl.ds(b0, BP), pl.ds(0, 1)], osem[b]).wait()


TR = (T + 7) // 8
BTILES = B // 128


def _repack_body(x_ref, o_ref):
    for i in range(TR):
        h = min(8, T - 8 * i)
        for j in range(BTILES):
            o_ref[0, i, j, 0:h, :] = x_ref[0, 8 * i:8 * i + h, 128 * j:128 * (j + 1)]


def _repack_idx(idx_ctb):
    return pl.pallas_call(
        _repack_body,
        grid=(C,),
        in_specs=[pl.BlockSpec((1, T, B), lambda c: (c, 0, 0))],
        out_specs=pl.BlockSpec((1, TR, BTILES, 8, 128), lambda c: (c, 0, 0, 0, 0)),
        out_shape=jax.ShapeDtypeStruct((C, TR, BTILES, 8, 128), jnp.int32),
    )(idx_ctb)


@jax.jit
def _embed_sum(idx_ctb, embed_table):
    mesh = plsc.VectorSubcoreMesh(core_axis_name="c", subcore_axis_name="s")
    return pl.kernel(
        _emb_body,
        out_type=jax.ShapeDtypeStruct((B, T, ED), jnp.float32),
        mesh=mesh,
        compiler_params=pltpu.CompilerParams(use_tc_tiling_on_sc=False),
        scratch_types=[
            pltpu.VMEM((C, BP), jnp.int32),
            pltpu.VMEM((C, BP), jnp.int32),
            pltpu.VMEM((C, BP, ED), jnp.float32),
            pltpu.VMEM((C, BP, ED), jnp.float32),
            pltpu.VMEM((BP, 1, ED), jnp.float32),
            pltpu.VMEM((BP, 1, ED), jnp.float32),
            pltpu.SemaphoreType.DMA,
            pltpu.SemaphoreType.DMA,
            pltpu.SemaphoreType.DMA,
            pltpu.SemaphoreType.DMA,
        ],
    )(idx_ctb, embed_table)


def kernel(ContTensor, CatTensor, LabelTensor, MaskTensor, DoseTensor, TimeDiffTensor, VTensor, VancoClTensor, PtList, LengList, embed_table):
    idx_ctb = _repack_idx(CatTensor.transpose(2, 1, 0))
    emb = _embed_sum(idx_ctb, embed_table)
    outEmb = jnp.concatenate((emb, ContTensor), axis=2)
    return (outEmb, LabelTensor, LengList, MaskTensor, DoseTensor, TimeDiffTensor, VTensor, VancoClTensor, PtList)
